# Initial kernel scaffold; baseline (speedup 1.0000x reference)
import jax
import jax.numpy as jnp
from jax.experimental import pallas as pl


def kernel(x, embedding):
    raise NotImplementedError("write your pallas kernel here")


# speedup vs baseline: 1.9298x; 1.9298x over previous
import jax
import jax.numpy as jnp
from jax.experimental import pallas as pl


def _add_kernel(x_ref, e_ref, o_ref):
    o_ref[...] = x_ref[...] + e_ref[...][None]


def kernel(x, embedding):
    B, T, D = x.shape
    TT = 256
    grid = (T // TT,)
    return pl.pallas_call(
        _add_kernel,
        grid=grid,
        in_specs=[
            pl.BlockSpec((B, TT, D), lambda i: (0, i, 0)),
            pl.BlockSpec((TT, D), lambda i: (i, 0)),
        ],
        out_specs=pl.BlockSpec((B, TT, D), lambda i: (0, i, 0)),
        out_shape=jax.ShapeDtypeStruct((B, T, D), x.dtype),
    )(x, embedding)


# speedup vs baseline: 1.9651x; 1.0183x over previous
import jax
import jax.numpy as jnp
from jax.experimental import pallas as pl


def _add_kernel(x_ref, e_ref, o_ref):
    o_ref[...] = x_ref[...] + e_ref[...][None]


def kernel(x, embedding):
    B, T, D = x.shape
    TT = 512
    grid = (T // TT,)
    return pl.pallas_call(
        _add_kernel,
        grid=grid,
        in_specs=[
            pl.BlockSpec((B, TT, D), lambda i: (0, i, 0)),
            pl.BlockSpec((TT, D), lambda i: (i, 0)),
        ],
        out_specs=pl.BlockSpec((B, TT, D), lambda i: (0, i, 0)),
        out_shape=jax.ShapeDtypeStruct((B, T, D), x.dtype),
    )(x, embedding)
